# Initial kernel scaffold; baseline (speedup 1.0000x reference)
import jax
import jax.numpy as jnp
from jax.experimental import pallas as pl


def kernel(bond_idx, bond_type_idx, num_nodes, batch_size, bond_weights):
    raise NotImplementedError("write your pallas kernel here")


# speedup vs baseline: 9.2926x; 9.2926x over previous
import functools

import jax
import jax.numpy as jnp
from jax import lax
from jax.experimental import pallas as pl
from jax.experimental.pallas import tpu as pltpu
from jax.experimental.pallas import tpu_sc as plsc

N = 128
FLAT = N * N
L = 16


def _sc_body(nb, e, ii_hbm, jj_hbm, tt_hbm, w_hbm, out_hbm,
             ivm, jvm, tvm, wvm, mat):
    wid = lax.axis_index("s") * 2 + lax.axis_index("c")
    base = wid * nb
    chunks = e // L

    pltpu.sync_copy(ii_hbm.at[pl.ds(base * e, nb * e)], ivm)
    pltpu.sync_copy(jj_hbm.at[pl.ds(base * e, nb * e)], jvm)
    pltpu.sync_copy(tt_hbm.at[pl.ds(base * e, nb * e)], tvm)
    pltpu.sync_copy(w_hbm, wvm)

    zeros = jnp.zeros((L,), jnp.float32)

    def zero_init(i, carry):
        mat[pl.ds(i * L, L)] = zeros
        return carry

    lax.fori_loop(0, FLAT // L, zero_init, 0)

    def batch_body(k, carry):
        off = k * e

        def pass_ij(c, carry):
            s = off + c * L
            iv = ivm[pl.ds(s, L)] + 1
            jv = jvm[pl.ds(s, L)] + 1
            wv = plsc.load_gather(wvm, [tvm[pl.ds(s, L)]])
            plsc.store_scatter(mat, [iv * N + jv], wv)
            return carry

        def pass_ji(c, carry):
            s = off + c * L
            iv = ivm[pl.ds(s, L)] + 1
            jv = jvm[pl.ds(s, L)] + 1
            wv = plsc.load_gather(wvm, [tvm[pl.ds(s, L)]])
            plsc.store_scatter(mat, [jv * N + iv], wv)
            return carry

        def pass_clear(c, carry):
            s = off + c * L
            iv = ivm[pl.ds(s, L)] + 1
            jv = jvm[pl.ds(s, L)] + 1
            plsc.store_scatter(mat, [iv * N + jv], zeros)
            plsc.store_scatter(mat, [jv * N + iv], zeros)
            return carry

        lax.fori_loop(0, chunks, pass_ij, 0)
        lax.fori_loop(0, chunks, pass_ji, 0)
        pltpu.sync_copy(mat, out_hbm.at[base + k])
        lax.fori_loop(0, chunks, pass_clear, 0)
        return carry

    lax.fori_loop(0, nb, batch_body, 0)


def kernel(bond_idx, bond_type_idx, num_nodes, batch_size, bond_weights):
    b, e = bond_type_idx.shape
    nw = 32
    nb = b // nw

    ii = bond_idx[..., 0].reshape(-1)
    jj = bond_idx[..., 1].reshape(-1)
    tt = bond_type_idx.reshape(-1)
    w16 = jnp.pad(bond_weights.astype(jnp.float32), (0, L - bond_weights.shape[0]))

    mesh = plsc.VectorSubcoreMesh(core_axis_name="c", subcore_axis_name="s")
    run = pl.kernel(
        functools.partial(_sc_body, nb, e),
        out_type=jax.ShapeDtypeStruct((b, FLAT), jnp.float32),
        mesh=mesh,
        compiler_params=pltpu.CompilerParams(needs_layout_passes=False),
        scratch_types=[
            pltpu.VMEM((nb * e,), jnp.int32),
            pltpu.VMEM((nb * e,), jnp.int32),
            pltpu.VMEM((nb * e,), jnp.int32),
            pltpu.VMEM((L,), jnp.float32),
            pltpu.VMEM((FLAT,), jnp.float32),
        ],
    )
    out = run(ii, jj, tt, w16)
    return out.reshape(b, N, N)


# speedup vs baseline: 15.0947x; 1.6244x over previous
import functools

import jax
import jax.numpy as jnp
from jax import lax
from jax.experimental import pallas as pl
from jax.experimental.pallas import tpu as pltpu
from jax.experimental.pallas import tpu_sc as plsc

N = 128
FLAT = N * N
L = 16
GB = 2


def _sc_body(nb, e, ii_hbm, jj_hbm, tt_hbm, w_hbm, out_hbm,
             ivm, jvm, tvm, wvm, buf0, buf1, sem0, sem1):
    wid = lax.axis_index("s") * 2 + lax.axis_index("c")
    base = wid * nb
    chunks = e // L
    bufs = (buf0, buf1)
    sems = (sem0, sem1)
    ngroups = nb // GB

    pltpu.sync_copy(ii_hbm.at[pl.ds(base * e, nb * e)], ivm)
    pltpu.sync_copy(jj_hbm.at[pl.ds(base * e, nb * e)], jvm)
    pltpu.sync_copy(tt_hbm.at[pl.ds(base * e, nb * e)], tvm)
    pltpu.sync_copy(w_hbm, wvm)

    zeros = jnp.zeros((L,), jnp.float32)

    def zero_init(i, carry):
        s = i * (8 * L)
        for buf in bufs:
            for u in range(8):
                buf[pl.ds(s + u * L, L)] = zeros
        return carry

    lax.fori_loop(0, GB * FLAT // (8 * L), zero_init, 0)

    def scatter_batch(k, buf, row):
        off = k * e

        rb = row * FLAT

        def pass_ij(c, carry):
            s = off + c * L
            iv = ivm[pl.ds(s, L)] + 1
            jv = jvm[pl.ds(s, L)] + 1
            wv = plsc.load_gather(wvm, [tvm[pl.ds(s, L)]])
            plsc.store_scatter(buf, [rb + iv * N + jv], wv)
            return carry

        def pass_ji(c, carry):
            s = off + c * L
            iv = ivm[pl.ds(s, L)] + 1
            jv = jvm[pl.ds(s, L)] + 1
            wv = plsc.load_gather(wvm, [tvm[pl.ds(s, L)]])
            plsc.store_scatter(buf, [rb + jv * N + iv], wv)
            return carry

        lax.fori_loop(0, chunks, pass_ij, 0)
        lax.fori_loop(0, chunks, pass_ji, 0)

    def clear_batch(k, buf, row):
        off = k * e

        rb = row * FLAT

        def pass_clear(c, carry):
            s = off + c * L
            iv = ivm[pl.ds(s, L)] + 1
            jv = jvm[pl.ds(s, L)] + 1
            plsc.store_scatter(buf, [rb + iv * N + jv], zeros)
            plsc.store_scatter(buf, [rb + jv * N + iv], zeros)
            return carry

        lax.fori_loop(0, chunks, pass_clear, 0)

    inflight = [None, None]
    for g in range(ngroups):
        slot = g % 2
        buf = bufs[slot]
        if inflight[slot] is not None:
            dma, gprev = inflight[slot]
            dma.wait()
            for r in range(GB):
                clear_batch(gprev * GB + r, buf, r)
        for r in range(GB):
            scatter_batch(g * GB + r, buf, r)
        dma = pltpu.async_copy(
            buf, out_hbm.at[pl.ds((base + g * GB) * FLAT, GB * FLAT)],
            sems[slot])
        inflight[slot] = (dma, g)
    for slot in range(2):
        if inflight[slot] is not None:
            inflight[slot][0].wait()


def kernel(bond_idx, bond_type_idx, num_nodes, batch_size, bond_weights):
    b, e = bond_type_idx.shape
    nw = 32
    nb = b // nw

    ii = bond_idx[..., 0].reshape(-1)
    jj = bond_idx[..., 1].reshape(-1)
    tt = bond_type_idx.reshape(-1)
    w16 = jnp.pad(bond_weights.astype(jnp.float32), (0, L - bond_weights.shape[0]))

    mesh = plsc.VectorSubcoreMesh(core_axis_name="c", subcore_axis_name="s")
    run = pl.kernel(
        functools.partial(_sc_body, nb, e),
        out_type=jax.ShapeDtypeStruct((b * FLAT,), jnp.float32),
        mesh=mesh,
        compiler_params=pltpu.CompilerParams(needs_layout_passes=False),
        scratch_types=[
            pltpu.VMEM((nb * e,), jnp.int32),
            pltpu.VMEM((nb * e,), jnp.int32),
            pltpu.VMEM((nb * e,), jnp.int32),
            pltpu.VMEM((L,), jnp.float32),
            pltpu.VMEM((GB * FLAT,), jnp.float32),
            pltpu.VMEM((GB * FLAT,), jnp.float32),
            pltpu.SemaphoreType.DMA,
            pltpu.SemaphoreType.DMA,
        ],
    )
    out = run(ii, jj, tt, w16)
    return out.reshape(b, N, N)
